# baseline (speedup 1.0000x reference)
import jax
import jax.numpy as jnp
from jax import lax
from jax.experimental import pallas as pl
from jax.experimental.pallas import tpu as pltpu

_ALPHA = 1.0
_BETA = 0.2
_DEPTH = 3
_BASETHETA = 1.0

N = 8192
BR = 64
NBUF = 8
NSTEPS = N // BR
ROUNDS = NSTEPS // NBUF


def _jacobi_coeffs(L):
    A_l = (2 * L + _ALPHA + _BETA) * (2 * L + _ALPHA + _BETA - 1) / (
        2 * L * (L + _ALPHA + _BETA))
    B_l = (2 * L + _ALPHA + _BETA - 1) * (_ALPHA ** 2 - _BETA ** 2) / (
        2 * L * (L + _ALPHA + _BETA) * (2 * L + _ALPHA + _BETA - 2))
    C_l = (L + _ALPHA - 1) * (L + _BETA - 1) * (2 * L + _ALPHA + _BETA) / (
        L * (L + _ALPHA + _BETA) * (2 * L + _ALPHA + _BETA - 2))
    return A_l, B_l, C_l


def _cubic_coeffs(th):
    p = 0.5 * (_ALPHA - _BETA)
    q = 0.5 * (_ALPHA + _BETA + 2.0)
    A2, B2, C2 = _jacobi_coeffs(2)
    A3, B3, C3 = _jacobi_coeffs(3)
    x2_0 = B2 * p - C2
    x2_1 = A2 * p + B2 * q
    x2_2 = A2 * q
    x3_0 = B3 * x2_0 - C3 * p
    x3_1 = A3 * x2_0 + B3 * x2_1 - C3 * q
    x3_2 = A3 * x2_1 + B3 * x2_2
    x3_3 = A3 * x2_2
    c0 = th[0] + th[1] * p + th[2] * x2_0 + th[3] * x3_0
    c1 = th[1] * q + th[2] * x2_1 + th[3] * x3_1
    c2 = th[2] * x2_2 + th[3] * x3_2
    c3 = th[3] * x3_3
    return c0, c1, c2, c3


def _body(th_ref, adj_hbm, out_hbm, inbuf, outbuf, insem, outsem):
    th = _BASETHETA * jnp.tanh(th_ref[0, :])
    c0, c1, c2, c3 = _cubic_coeffs(th)

    def in_copy(step, b):
        return pltpu.make_async_copy(
            adj_hbm.at[pl.ds(step * BR, BR), :], inbuf.at[b], insem.at[b])

    def out_copy(step, b):
        return pltpu.make_async_copy(
            outbuf.at[b], out_hbm.at[pl.ds(step * BR, BR), :], outsem.at[b])

    for b in range(NBUF):
        in_copy(b, b).start()

    def round_step(r, carry):
        for b in range(NBUF):
            step = r * NBUF + b
            in_copy(step, b).wait()

            @pl.when(r > 0)
            def _():
                out_copy(step, b).wait()

            a = inbuf[b]
            outbuf[b] = c0 + a * (c1 + a * (c2 + a * c3))
            out_copy(step, b).start()

            @pl.when(step + NBUF < NSTEPS)
            def _():
                in_copy(step + NBUF, b).start()
        return carry

    lax.fori_loop(0, ROUNDS, round_step, 0)

    for b in range(NBUF):
        out_copy((ROUNDS - 1) * NBUF + b, b).wait()


def kernel(adj, thetas):
    th2d = thetas.reshape(1, _DEPTH + 1)
    return pl.pallas_call(
        _body,
        grid=(),
        in_specs=[
            pl.BlockSpec(memory_space=pltpu.VMEM),
            pl.BlockSpec(memory_space=pl.ANY),
        ],
        out_specs=pl.BlockSpec(memory_space=pl.ANY),
        out_shape=jax.ShapeDtypeStruct((N, N), jnp.float32),
        scratch_shapes=[
            pltpu.VMEM((NBUF, BR, N), jnp.float32),
            pltpu.VMEM((NBUF, BR, N), jnp.float32),
            pltpu.SemaphoreType.DMA((NBUF,)),
            pltpu.SemaphoreType.DMA((NBUF,)),
        ],
    )(th2d, adj)


# speedup vs baseline: 1.0007x; 1.0007x over previous
import jax
import jax.numpy as jnp
from jax import lax
from jax.experimental import pallas as pl
from jax.experimental.pallas import tpu as pltpu

_ALPHA = 1.0
_BETA = 0.2
_DEPTH = 3
_BASETHETA = 1.0

N = 8192
BR = 128
NBUF = 4
NSTEPS = N // BR
ROUNDS = NSTEPS // NBUF


def _jacobi_coeffs(L):
    A_l = (2 * L + _ALPHA + _BETA) * (2 * L + _ALPHA + _BETA - 1) / (
        2 * L * (L + _ALPHA + _BETA))
    B_l = (2 * L + _ALPHA + _BETA - 1) * (_ALPHA ** 2 - _BETA ** 2) / (
        2 * L * (L + _ALPHA + _BETA) * (2 * L + _ALPHA + _BETA - 2))
    C_l = (L + _ALPHA - 1) * (L + _BETA - 1) * (2 * L + _ALPHA + _BETA) / (
        L * (L + _ALPHA + _BETA) * (2 * L + _ALPHA + _BETA - 2))
    return A_l, B_l, C_l


def _cubic_coeffs(th):
    p = 0.5 * (_ALPHA - _BETA)
    q = 0.5 * (_ALPHA + _BETA + 2.0)
    A2, B2, C2 = _jacobi_coeffs(2)
    A3, B3, C3 = _jacobi_coeffs(3)
    x2_0 = B2 * p - C2
    x2_1 = A2 * p + B2 * q
    x2_2 = A2 * q
    x3_0 = B3 * x2_0 - C3 * p
    x3_1 = A3 * x2_0 + B3 * x2_1 - C3 * q
    x3_2 = A3 * x2_1 + B3 * x2_2
    x3_3 = A3 * x2_2
    c0 = th[0] + th[1] * p + th[2] * x2_0 + th[3] * x3_0
    c1 = th[1] * q + th[2] * x2_1 + th[3] * x3_1
    c2 = th[2] * x2_2 + th[3] * x3_2
    c3 = th[3] * x3_3
    return c0, c1, c2, c3


def _body(th_ref, adj_hbm, out_hbm, inbuf, outbuf, insem, outsem):
    th = _BASETHETA * jnp.tanh(th_ref[0, :])
    c0, c1, c2, c3 = _cubic_coeffs(th)

    H = BR // 2

    def in_copy(step, b, h):
        return pltpu.make_async_copy(
            adj_hbm.at[pl.ds(step * BR + h * H, H), :],
            inbuf.at[b, pl.ds(h * H, H)], insem.at[b, h])

    def out_copy(step, b, h):
        return pltpu.make_async_copy(
            outbuf.at[b, pl.ds(h * H, H)],
            out_hbm.at[pl.ds(step * BR + h * H, H), :], outsem.at[b, h])

    for b in range(NBUF):
        in_copy(b, b, 0).start()
        in_copy(b, b, 1).start()

    def round_step(r, carry):
        for b in range(NBUF):
            step = r * NBUF + b
            in_copy(step, b, 0).wait()
            in_copy(step, b, 1).wait()

            @pl.when(r > 0)
            def _():
                out_copy(step, b, 0).wait()
                out_copy(step, b, 1).wait()

            a = inbuf[b]
            outbuf[b] = c0 + a * (c1 + a * (c2 + a * c3))
            out_copy(step, b, 0).start()
            out_copy(step, b, 1).start()

            @pl.when(step + NBUF < NSTEPS)
            def _():
                in_copy(step + NBUF, b, 0).start()
                in_copy(step + NBUF, b, 1).start()
        return carry

    lax.fori_loop(0, ROUNDS, round_step, 0)

    for b in range(NBUF):
        out_copy((ROUNDS - 1) * NBUF + b, b, 0).wait()
        out_copy((ROUNDS - 1) * NBUF + b, b, 1).wait()


def kernel(adj, thetas):
    th2d = thetas.reshape(1, _DEPTH + 1)
    return pl.pallas_call(
        _body,
        grid=(),
        in_specs=[
            pl.BlockSpec(memory_space=pltpu.VMEM),
            pl.BlockSpec(memory_space=pl.ANY),
        ],
        out_specs=pl.BlockSpec(memory_space=pl.ANY),
        out_shape=jax.ShapeDtypeStruct((N, N), jnp.float32),
        scratch_shapes=[
            pltpu.VMEM((NBUF, BR, N), jnp.float32),
            pltpu.VMEM((NBUF, BR, N), jnp.float32),
            pltpu.SemaphoreType.DMA((NBUF, 2)),
            pltpu.SemaphoreType.DMA((NBUF, 2)),
        ],
    )(th2d, adj)
